# Initial kernel scaffold; baseline (speedup 1.0000x reference)
import jax
import jax.numpy as jnp
from jax.experimental import pallas as pl


def kernel(points, indices):
    raise NotImplementedError("write your pallas kernel here")


# speedup vs baseline: 1.4608x; 1.4608x over previous
import functools

import jax
import jax.numpy as jnp
from jax import lax
from jax.experimental import pallas as pl
from jax.experimental.pallas import tpu as pltpu
from jax.experimental.pallas import tpu_sc as plsc

NC = 2
NS = 16
NW = NC * NS
L = 16

K = 32
D = 128
G = 8
GK = G * K
CH = GK // 128
NCHUNK = D // L


def _pool_body(points_hbm, idx_hbm, out_hbm, idx_v, rows_v, out_v, sem0, sem1,
               *, nb):
    sems = (sem0, sem1)
    wid = lax.axis_index("s") * NC + lax.axis_index("c")
    row_base = wid * (nb * G)
    idx_base = wid * (nb * CH)

    def copy_idx(batch, buf):
        pltpu.sync_copy(idx_hbm.at[pl.ds(idx_base + batch * CH, CH)],
                        idx_v.at[buf])

    def fire(buf):
        for c in range(CH):
            pltpu.async_copy(points_hbm.at[idx_v.at[buf, c]],
                             rows_v.at[buf, pl.ds(c * 128, 128)],
                             sems[buf])

    def wait(buf):
        for c in range(CH):
            pltpu.make_async_copy(points_hbm.at[idx_v.at[buf, c]],
                                  rows_v.at[buf, pl.ds(c * 128, 128)],
                                  sems[buf]).wait()

    def compute(batch, buf):
        rv = rows_v.at[buf]

        def per_row(g, carry):
            r0 = g * K
            accs = tuple(rv[r0, pl.ds(c * L, L)] for c in range(NCHUNK))

            def jstep(j, accs):
                r = r0 + j
                return tuple(
                    jnp.maximum(accs[c], rv[r, pl.ds(c * L, L)])
                    for c in range(NCHUNK))

            accs = lax.fori_loop(1, K, jstep, accs)
            for c in range(NCHUNK):
                out_v[g, pl.ds(c * L, L)] = accs[c]
            return carry

        lax.fori_loop(0, G, per_row, 0)
        pltpu.sync_copy(out_v, out_hbm.at[pl.ds(row_base + batch * G, G)])

    copy_idx(0, 0)
    fire(0)

    def two_batches(t, carry):
        for b in range(2):
            i = 2 * t + b
            nxt = i + 1
            nbuf = (b + 1) % 2

            @pl.when(nxt < nb)
            def _():
                copy_idx(nxt, nbuf)
                fire(nbuf)

            wait(b)
            compute(i, b)
        return carry

    lax.fori_loop(0, nb // 2, two_batches, 0)


def kernel(points, indices):
    m, k = indices.shape
    n, d = points.shape
    assert k == K and d == D

    rows_per_w = -(-m // (NW * G)) * G
    nb = rows_per_w // G
    if nb % 2:
        nb += 1
        rows_per_w += G
    m_pad = NW * rows_per_w

    idx = indices.astype(jnp.int32)
    idx = jnp.pad(idx, ((0, m_pad - m), (0, 0)))
    idx2 = idx.reshape(m_pad * K // 128, 128)

    pool = functools.partial(
        pl.kernel,
        out_type=jax.ShapeDtypeStruct((m_pad, D), jnp.float32),
        mesh=plsc.VectorSubcoreMesh(core_axis_name="c", subcore_axis_name="s"),
        scratch_types=[
            pltpu.VMEM((2, CH, 128), jnp.int32),
            pltpu.VMEM((2, GK, D), jnp.float32),
            pltpu.VMEM((G, D), jnp.float32),
            pltpu.SemaphoreType.DMA,
            pltpu.SemaphoreType.DMA,
        ],
    )(functools.partial(_pool_body, nb=nb))

    out = pool(points, idx2)
    return out[:m]
